# baseline (speedup 1.0000x reference)
import functools

import jax
import jax.numpy as jnp
from jax import lax
from jax.experimental import pallas as pl
from jax.experimental.pallas import tpu as pltpu
from jax.experimental.pallas import tpu_sc as plsc

N = 10000
R = 4
G = 64
DH = 512
NC = 2
NS = 16
CHUNK = 128
SEGS = 10240
S_PAD = R * SEGS
TS = S_PAD // NS
E_PAD = 163840
NSLOT = 5
QW1 = E_PAD // (NC * NS) // 512
QW2 = E_PAD // NS // 512
NCH1 = QW1 * 4
NCH2 = QW2 * 4
BN = 1000


def _stage_edges(src_h, dst_h, et_h, qbuf, srcq, segq, nquads, q0, src_mult):
  def pre_body(q, _):
    base = (q0 + q) * 512
    pltpu.sync_copy(src_h.at[pl.ds(base, 512)], qbuf.at[0])
    pltpu.sync_copy(dst_h.at[pl.ds(base, 512)], qbuf.at[1])
    pltpu.sync_copy(et_h.at[pl.ds(base, 512)], qbuf.at[2])
    for u in range(4):
      for j in range(CHUNK // 16):
        sl = pl.ds(u * CHUNK + j * 16, 16)
        dl = pl.ds(j * 16, 16)
        srcq[q * 4 + u, dl] = qbuf[0, sl] * src_mult
        segq[q * 4 + u, dl] = qbuf[2, sl] * SEGS + qbuf[1, sl]
    return 0
  lax.fori_loop(0, nquads, pre_body, 0)


def _zero_acc(zero_v, acc, s, zsem):
  for i in range(TS // CHUNK):
    pltpu.async_copy(zero_v, acc.at[pl.ds(s * TS + i * CHUNK, CHUNK)], zsem)
  for i in range(TS // CHUNK):
    pltpu.make_async_copy(zero_v,
                          acc.at[pl.ds(s * TS, CHUNK)], zsem).wait()


def _sc_agg1(xp, srcp, dstp, etp):
  mesh = plsc.VectorSubcoreMesh(core_axis_name="c", subcore_axis_name="s")

  @functools.partial(
      pl.kernel,
      out_type=jax.ShapeDtypeStruct((NC, S_PAD, 16), jnp.float32),
      mesh=mesh,
      compiler_params=pltpu.CompilerParams(use_tc_tiling_on_sc=False),
      scratch_types=[
          pltpu.VMEM((3, 512), jnp.int32),
          pltpu.VMEM((NCH1, CHUNK), jnp.int32),
          pltpu.VMEM((NCH1, CHUNK), jnp.int32),
          pltpu.VMEM((NSLOT, CHUNK, 16), jnp.float32),
          pltpu.VMEM((CHUNK, 16), jnp.float32),
          pltpu.VMEM_SHARED((S_PAD, 16), jnp.float32),
          pltpu.SemaphoreType.DMA((NSLOT,)),
          pltpu.SemaphoreType.DMA((NSLOT,)),
          pltpu.SemaphoreType.DMA,
      ],
  )
  def kfn(xp_h, src_h, dst_h, et_h, out_h,
          qbuf, srcq, segq, rows_v, zero_v, acc, gsem, ssem, zsem):
    c = lax.axis_index("c")
    s = lax.axis_index("s")
    w = c * NS + s
    z16 = jnp.zeros((16,), jnp.float32)

    def zrow(i, _):
      zero_v[i, :] = z16
      return 0
    lax.fori_loop(0, CHUNK, zrow, 0)

    _stage_edges(src_h, dst_h, et_h, qbuf, srcq, segq, QW1, w * QW1, 1)
    _zero_acc(zero_v, acc, s, zsem)
    plsc.subcore_barrier()

    def scatter_wait(t):
      pltpu.make_async_copy(rows_v.at[t], acc.at[segq.at[0]],
                            ssem.at[t]).wait()

    def emit_group(g, first):
      gds = []
      for t in range(NSLOT):
        k = g * NSLOT + t
        if not first:
          scatter_wait(t)
        gds.append(pltpu.async_copy(xp_h.at[srcq.at[k]], rows_v.at[t],
                                    gsem.at[t]))
      for t in range(NSLOT):
        k = g * NSLOT + t
        gds[t].wait()
        pltpu.async_copy(rows_v.at[t], acc.at[segq.at[k]], ssem.at[t],
                         add=True)

    emit_group(0, True)

    def gloop(g, _):
      emit_group(g, False)
      return 0
    lax.fori_loop(1, NCH1 // NSLOT, gloop, 0)
    for t in range(NSLOT):
      scatter_wait(t)
    plsc.subcore_barrier()

    pltpu.sync_copy(acc.at[pl.ds(s * TS, TS)],
                    out_h.at[c, pl.ds(s * TS, TS)])

  return kfn(xp, srcp, dstp, etp)


def _sc_agg2(hflat, srcp, dstp, etp):
  mesh = plsc.VectorSubcoreMesh(core_axis_name="c", subcore_axis_name="s")

  @functools.partial(
      pl.kernel,
      out_type=jax.ShapeDtypeStruct((S_PAD, DH), jnp.float32),
      mesh=mesh,
      compiler_params=pltpu.CompilerParams(use_tc_tiling_on_sc=False),
      scratch_types=[
          pltpu.VMEM((3, 512), jnp.int32),
          pltpu.VMEM((NCH2, CHUNK), jnp.int32),
          pltpu.VMEM((NCH2, CHUNK), jnp.int32),
          pltpu.VMEM((NSLOT, CHUNK), jnp.int32),
          pltpu.VMEM((NSLOT, CHUNK, 32), jnp.float32),
          pltpu.VMEM((CHUNK, 32), jnp.float32),
          pltpu.VMEM_SHARED((S_PAD, 32), jnp.float32),
          pltpu.SemaphoreType.DMA((NSLOT,)),
          pltpu.SemaphoreType.DMA((NSLOT,)),
          pltpu.SemaphoreType.DMA,
      ],
  )
  def kfn(h_h, src_h, dst_h, et_h, out_h,
          qbuf, idx0q, segq, idxb, rows_v, zero_v, acc, gsem, ssem, zsem):
    c = lax.axis_index("c")
    s = lax.axis_index("s")
    z16 = jnp.zeros((16,), jnp.float32)

    def zrow(i, _):
      zero_v[i, pl.ds(0, 16)] = z16
      zero_v[i, pl.ds(16, 16)] = z16
      return 0
    lax.fori_loop(0, CHUNK, zrow, 0)

    _stage_edges(src_h, dst_h, et_h, qbuf, idx0q, segq, QW2, s * QW2, 16)

    def scatter_wait(t):
      pltpu.make_async_copy(rows_v.at[t], acc.at[segq.at[0]],
                            ssem.at[t]).wait()

    def jslice_body(jsl, _):
      b = c * 8 + jsl

      _zero_acc(zero_v, acc, s, zsem)
      plsc.subcore_barrier()

      def emit_group(g, first):
        gds = []
        for t in range(NSLOT):
          k = g * NSLOT + t
          if not first:
            scatter_wait(t)
          for j in range(CHUNK // 16):
            sl = pl.ds(j * 16, 16)
            idxb[t, sl] = idx0q[k, sl] + b
          gds.append(pltpu.async_copy(h_h.at[idxb.at[t]], rows_v.at[t],
                                      gsem.at[t]))
        for t in range(NSLOT):
          k = g * NSLOT + t
          gds[t].wait()
          pltpu.async_copy(rows_v.at[t], acc.at[segq.at[k]], ssem.at[t],
                           add=True)

      emit_group(0, True)

      def gloop(g, _):
        emit_group(g, False)
        return 0
      lax.fori_loop(1, NCH2 // NSLOT, gloop, 0)

      for t in range(NSLOT):
        scatter_wait(t)
      plsc.subcore_barrier()

      pltpu.sync_copy(acc.at[pl.ds(s * TS, TS)],
                      out_h.at[pl.ds(s * TS, TS), pl.ds(b * 32, 32)])
      plsc.subcore_barrier()
      return 0

    lax.fori_loop(0, 8, jslice_body, 0)

  return kfn(hflat, srcp, dstp, etp)


def _tc_layer1(a1p, xp, w1p, root1p, b1r):
  nb = N // BN

  def body(a1_ref, x_ref, w1_ref, root_ref, b1_ref, h_ref, inv_ref):
    a = a1_ref[0] + a1_ref[1]
    cnt = a[:, :, 15]
    inv = 1.0 / jnp.maximum(cnt, 1.0)
    mean = a * inv[:, :, None]
    h = jnp.dot(x_ref[...], root_ref[...],
                preferred_element_type=jnp.float32) + b1_ref[...]
    for r in range(R):
      h += jnp.dot(mean[r], w1_ref[r], preferred_element_type=jnp.float32)
    h_ref[...] = jnp.maximum(h, 0.0)
    inv_ref[...] = inv[None]

  return pl.pallas_call(
      body,
      grid=(nb,),
      in_specs=[
          pl.BlockSpec((2, R, BN, 16), lambda i: (0, 0, i, 0)),
          pl.BlockSpec((BN, 16), lambda i: (i, 0)),
          pl.BlockSpec((R, 16, DH), lambda i: (0, 0, 0)),
          pl.BlockSpec((16, DH), lambda i: (0, 0)),
          pl.BlockSpec((1, DH), lambda i: (0, 0)),
      ],
      out_specs=[
          pl.BlockSpec((BN, DH), lambda i: (i, 0)),
          pl.BlockSpec((1, R, BN), lambda i: (i, 0, 0)),
      ],
      out_shape=[
          jax.ShapeDtypeStruct((N, DH), jnp.float32),
          jax.ShapeDtypeStruct((N // BN, R, BN), jnp.float32),
      ],
  )(a1p, xp, w1p, root1p, b1r)


def _tc_layer2(a2, invr, h, w2, root2, b2r, batch3d):
  nb = N // BN

  def body(a2_ref, inv_ref, h_ref, w2_ref, root_ref, b2_ref, batch_ref,
           out_ref, pool, cnt):
    i = pl.program_id(0)
    inv = inv_ref[0]
    z = jnp.dot(h_ref[...], root_ref[...],
                preferred_element_type=jnp.float32) + b2_ref[...]
    for r in range(R):
      mean = a2_ref[r] * inv[r][:, None]
      z += jnp.dot(mean, w2_ref[r], preferred_element_type=jnp.float32)
    z = jnp.maximum(z, 0.0)

    gids = lax.broadcasted_iota(jnp.int32, (G, BN), 0)
    oh = (batch_ref[0] == gids).astype(jnp.float32)
    psum = jnp.dot(oh, z, preferred_element_type=jnp.float32)
    csum = jnp.sum(oh, axis=1)[:, None]

    @pl.when(i == 0)
    def _():
      pool[...] = psum
      cnt[...] = csum

    @pl.when(i > 0)
    def _():
      pool[...] += psum
      cnt[...] += csum

    out_ref[...] = pool[...] / jnp.maximum(cnt[...], 1.0)

  return pl.pallas_call(
      body,
      grid=(nb,),
      in_specs=[
          pl.BlockSpec((R, BN, DH), lambda i: (0, i, 0)),
          pl.BlockSpec((1, R, BN), lambda i: (i, 0, 0)),
          pl.BlockSpec((BN, DH), lambda i: (i, 0)),
          pl.BlockSpec((R, DH, DH), lambda i: (0, 0, 0)),
          pl.BlockSpec((DH, DH), lambda i: (0, 0)),
          pl.BlockSpec((1, DH), lambda i: (0, 0)),
          pl.BlockSpec((1, 1, BN), lambda i: (i, 0, 0)),
      ],
      out_specs=pl.BlockSpec((G, DH), lambda i: (0, 0)),
      out_shape=jax.ShapeDtypeStruct((G, DH), jnp.float32),
      scratch_shapes=[
          pltpu.VMEM((G, DH), jnp.float32),
          pltpu.VMEM((G, 1), jnp.float32),
      ],
  )(a2, invr, h, w2, root2, b2r, batch3d)


def kernel(x, edge_index, edge_type, batch, W1, root1, b1, W2, root2, b2):
  e = edge_index.shape[1]
  pad = E_PAD - e
  src = edge_index[0].astype(jnp.int32)
  dst = edge_index[1].astype(jnp.int32)
  et = edge_type.astype(jnp.int32)
  srcp = jnp.concatenate([src, jnp.zeros((pad,), jnp.int32)])
  dstp = jnp.concatenate([dst, jnp.full((pad,), N, jnp.int32)])
  etp = jnp.concatenate([et, jnp.zeros((pad,), jnp.int32)])
  xp = jnp.concatenate([x, jnp.ones((N, 1), jnp.float32)], axis=1)
  w1p = jnp.pad(W1, ((0, 0), (0, 1), (0, 0)))
  root1p = jnp.pad(root1, ((0, 1), (0, 0)))
  b1r = b1[None, :]
  b2r = b2[None, :]
  batch3d = batch.astype(jnp.int32).reshape(N // BN, 1, BN)

  a1p = _sc_agg1(xp, srcp, dstp, etp).reshape(NC, R, SEGS, 16)
  h, invr = _tc_layer1(a1p, xp, w1p, root1p, b1r)
  hflat = h.reshape(N * 16, 32)
  a2 = _sc_agg2(hflat, srcp, dstp, etp).reshape(R, SEGS, DH)
  return _tc_layer2(a2, invr, h, W2, root2, b2r, batch3d)


# speedup vs baseline: 1.0908x; 1.0908x over previous
import functools

import jax
import jax.numpy as jnp
from jax import lax
from jax.experimental import pallas as pl
from jax.experimental.pallas import tpu as pltpu
from jax.experimental.pallas import tpu_sc as plsc

N = 10000
R = 4
G = 64
DH = 512
NC = 2
NS = 16
CHUNK = 128
SEGS = 10240
S_PAD = R * SEGS
TS = S_PAD // NS
E_PAD = 163840
NSLOT = 5
QW1 = E_PAD // (NC * NS) // 512
QW2 = E_PAD // NS // 512
NCH1 = QW1 * 4
NCH2 = QW2 * 4
BN = 1000


def _stage_edges(ed_h, qbuf, srcq, segq, nquads, q0, src_mult):
  def pre_body(q, _):
    pltpu.sync_copy(ed_h.at[q0 + q], qbuf)
    for u in range(4):
      for j in range(CHUNK // 16):
        sl = pl.ds(u * CHUNK + j * 16, 16)
        dl = pl.ds(j * 16, 16)
        srcq[q * 4 + u, dl] = qbuf[0, sl] * src_mult
        segq[q * 4 + u, dl] = qbuf[2, sl] * SEGS + qbuf[1, sl]
    return 0
  lax.fori_loop(0, nquads, pre_body, 0)


def _zero_acc(zero_v, acc, s, zsem):
  for i in range(TS // CHUNK):
    pltpu.async_copy(zero_v, acc.at[pl.ds(s * TS + i * CHUNK, CHUNK)], zsem)
  for i in range(TS // CHUNK):
    pltpu.make_async_copy(zero_v,
                          acc.at[pl.ds(s * TS, CHUNK)], zsem).wait()


def _sc_agg1(xp, edch):
  mesh = plsc.VectorSubcoreMesh(core_axis_name="c", subcore_axis_name="s")

  @functools.partial(
      pl.kernel,
      out_type=jax.ShapeDtypeStruct((NC, S_PAD, 16), jnp.float32),
      mesh=mesh,
      compiler_params=pltpu.CompilerParams(use_tc_tiling_on_sc=False),
      scratch_types=[
          pltpu.VMEM((3, 512), jnp.int32),
          pltpu.VMEM((NCH1, CHUNK), jnp.int32),
          pltpu.VMEM((NCH1, CHUNK), jnp.int32),
          pltpu.VMEM((NSLOT, CHUNK, 16), jnp.float32),
          pltpu.VMEM((CHUNK, 16), jnp.float32),
          pltpu.VMEM_SHARED((S_PAD, 16), jnp.float32),
          pltpu.SemaphoreType.DMA((NSLOT,)),
          pltpu.SemaphoreType.DMA((NSLOT,)),
          pltpu.SemaphoreType.DMA,
      ],
  )
  def kfn(xp_h, ed_h, out_h,
          qbuf, srcq, segq, rows_v, zero_v, acc, gsem, ssem, zsem):
    c = lax.axis_index("c")
    s = lax.axis_index("s")
    w = c * NS + s
    z16 = jnp.zeros((16,), jnp.float32)

    def zrow(i, _):
      zero_v[i, :] = z16
      return 0
    lax.fori_loop(0, CHUNK, zrow, 0)

    _stage_edges(ed_h, qbuf, srcq, segq, QW1, w * QW1, 1)
    _zero_acc(zero_v, acc, s, zsem)
    plsc.subcore_barrier()

    def scatter_wait(t):
      pltpu.make_async_copy(rows_v.at[t], acc.at[segq.at[0]],
                            ssem.at[t]).wait()

    def emit_group(g, first):
      gds = []
      for t in range(NSLOT):
        k = g * NSLOT + t
        if not first:
          scatter_wait(t)
        gds.append(pltpu.async_copy(xp_h.at[srcq.at[k]], rows_v.at[t],
                                    gsem.at[t]))
      for t in range(NSLOT):
        k = g * NSLOT + t
        gds[t].wait()
        pltpu.async_copy(rows_v.at[t], acc.at[segq.at[k]], ssem.at[t],
                         add=True)

    emit_group(0, True)

    def gloop(g, _):
      emit_group(g, False)
      return 0
    lax.fori_loop(1, NCH1 // NSLOT, gloop, 0)
    for t in range(NSLOT):
      scatter_wait(t)
    plsc.subcore_barrier()

    pltpu.sync_copy(acc.at[pl.ds(s * TS, TS)],
                    out_h.at[c, pl.ds(s * TS, TS)])

  return kfn(xp, edch)


def _sc_agg2(hflat, edch):
  mesh = plsc.VectorSubcoreMesh(core_axis_name="c", subcore_axis_name="s")

  @functools.partial(
      pl.kernel,
      out_type=jax.ShapeDtypeStruct((S_PAD, DH), jnp.float32),
      mesh=mesh,
      compiler_params=pltpu.CompilerParams(use_tc_tiling_on_sc=False),
      scratch_types=[
          pltpu.VMEM((3, 512), jnp.int32),
          pltpu.VMEM((NCH2, CHUNK), jnp.int32),
          pltpu.VMEM((NCH2, CHUNK), jnp.int32),
          pltpu.VMEM((NSLOT, CHUNK), jnp.int32),
          pltpu.VMEM((NSLOT, CHUNK, 32), jnp.float32),
          pltpu.VMEM((CHUNK, 32), jnp.float32),
          pltpu.VMEM_SHARED((S_PAD, 32), jnp.float32),
          pltpu.SemaphoreType.DMA((NSLOT,)),
          pltpu.SemaphoreType.DMA((NSLOT,)),
          pltpu.SemaphoreType.DMA,
      ],
  )
  def kfn(h_h, ed_h, out_h,
          qbuf, idx0q, segq, idxb, rows_v, zero_v, acc, gsem, ssem, zsem):
    c = lax.axis_index("c")
    s = lax.axis_index("s")
    z16 = jnp.zeros((16,), jnp.float32)

    def zrow(i, _):
      zero_v[i, pl.ds(0, 16)] = z16
      zero_v[i, pl.ds(16, 16)] = z16
      return 0
    lax.fori_loop(0, CHUNK, zrow, 0)

    _stage_edges(ed_h, qbuf, idx0q, segq, QW2, s * QW2, 16)

    def scatter_wait(t):
      pltpu.make_async_copy(rows_v.at[t], acc.at[segq.at[0]],
                            ssem.at[t]).wait()

    def jslice_body(jsl, _):
      b = c * 8 + jsl

      _zero_acc(zero_v, acc, s, zsem)
      plsc.subcore_barrier()

      def emit_group(g, first):
        gds = []
        for t in range(NSLOT):
          k = g * NSLOT + t
          if not first:
            scatter_wait(t)
          for j in range(CHUNK // 16):
            sl = pl.ds(j * 16, 16)
            idxb[t, sl] = idx0q[k, sl] + b
          gds.append(pltpu.async_copy(h_h.at[idxb.at[t]], rows_v.at[t],
                                      gsem.at[t]))
        for t in range(NSLOT):
          k = g * NSLOT + t
          gds[t].wait()
          pltpu.async_copy(rows_v.at[t], acc.at[segq.at[k]], ssem.at[t],
                           add=True)

      emit_group(0, True)

      def gloop(g, _):
        emit_group(g, False)
        return 0
      lax.fori_loop(1, NCH2 // NSLOT, gloop, 0)

      for t in range(NSLOT):
        scatter_wait(t)
      plsc.subcore_barrier()

      pltpu.sync_copy(acc.at[pl.ds(s * TS, TS)],
                      out_h.at[pl.ds(s * TS, TS), pl.ds(b * 32, 32)])
      plsc.subcore_barrier()
      return 0

    lax.fori_loop(0, 8, jslice_body, 0)

  return kfn(hflat, edch)


def _tc_layer1(a1p, xp, w1p, root1p, b1r):
  nb = N // BN

  def body(a1_ref, x_ref, w1_ref, root_ref, b1_ref, h_ref, inv_ref):
    a = a1_ref[0] + a1_ref[1]
    cnt = a[:, :, 15]
    inv = 1.0 / jnp.maximum(cnt, 1.0)
    mean = a * inv[:, :, None]
    h = jnp.dot(x_ref[...], root_ref[...],
                preferred_element_type=jnp.float32) + b1_ref[...]
    for r in range(R):
      h += jnp.dot(mean[r], w1_ref[r], preferred_element_type=jnp.float32)
    h_ref[...] = jnp.maximum(h, 0.0)
    inv_ref[...] = inv[None]

  return pl.pallas_call(
      body,
      grid=(nb,),
      in_specs=[
          pl.BlockSpec((2, R, BN, 16), lambda i: (0, 0, i, 0)),
          pl.BlockSpec((BN, 16), lambda i: (i, 0)),
          pl.BlockSpec((R, 16, DH), lambda i: (0, 0, 0)),
          pl.BlockSpec((16, DH), lambda i: (0, 0)),
          pl.BlockSpec((1, DH), lambda i: (0, 0)),
      ],
      out_specs=[
          pl.BlockSpec((BN, DH), lambda i: (i, 0)),
          pl.BlockSpec((1, R, BN), lambda i: (i, 0, 0)),
      ],
      out_shape=[
          jax.ShapeDtypeStruct((N, DH), jnp.float32),
          jax.ShapeDtypeStruct((N // BN, R, BN), jnp.float32),
      ],
  )(a1p, xp, w1p, root1p, b1r)


def _tc_layer2(a2, invr, h, w2, root2, b2r, batch3d):
  nb = N // BN

  def body(a2_ref, inv_ref, h_ref, w2_ref, root_ref, b2_ref, batch_ref,
           out_ref, pool, cnt):
    i = pl.program_id(0)
    inv = inv_ref[0]
    z = jnp.dot(h_ref[...], root_ref[...],
                preferred_element_type=jnp.float32) + b2_ref[...]
    for r in range(R):
      mean = a2_ref[r] * inv[r][:, None]
      z += jnp.dot(mean, w2_ref[r], preferred_element_type=jnp.float32)
    z = jnp.maximum(z, 0.0)

    gids = lax.broadcasted_iota(jnp.int32, (G, BN), 0)
    oh = (batch_ref[0] == gids).astype(jnp.float32)
    psum = jnp.dot(oh, z, preferred_element_type=jnp.float32)
    csum = jnp.sum(oh, axis=1)[:, None]

    @pl.when(i == 0)
    def _():
      pool[...] = psum
      cnt[...] = csum

    @pl.when(i > 0)
    def _():
      pool[...] += psum
      cnt[...] += csum

    out_ref[...] = pool[...] / jnp.maximum(cnt[...], 1.0)

  return pl.pallas_call(
      body,
      grid=(nb,),
      in_specs=[
          pl.BlockSpec((R, BN, DH), lambda i: (0, i, 0)),
          pl.BlockSpec((1, R, BN), lambda i: (i, 0, 0)),
          pl.BlockSpec((BN, DH), lambda i: (i, 0)),
          pl.BlockSpec((R, DH, DH), lambda i: (0, 0, 0)),
          pl.BlockSpec((DH, DH), lambda i: (0, 0)),
          pl.BlockSpec((1, DH), lambda i: (0, 0)),
          pl.BlockSpec((1, 1, BN), lambda i: (i, 0, 0)),
      ],
      out_specs=pl.BlockSpec((G, DH), lambda i: (0, 0)),
      out_shape=jax.ShapeDtypeStruct((G, DH), jnp.float32),
      scratch_shapes=[
          pltpu.VMEM((G, DH), jnp.float32),
          pltpu.VMEM((G, 1), jnp.float32),
      ],
  )(a2, invr, h, w2, root2, b2r, batch3d)


def kernel(x, edge_index, edge_type, batch, W1, root1, b1, W2, root2, b2):
  e = edge_index.shape[1]
  pad = E_PAD - e
  src = edge_index[0].astype(jnp.int32)
  dst = edge_index[1].astype(jnp.int32)
  et = edge_type.astype(jnp.int32)
  srcp = jnp.concatenate([src, jnp.zeros((pad,), jnp.int32)])
  dstp = jnp.concatenate([dst, jnp.full((pad,), N, jnp.int32)])
  etp = jnp.concatenate([et, jnp.zeros((pad,), jnp.int32)])
  edch = jnp.stack([srcp, dstp, etp]).reshape(3, E_PAD // 512, 512)
  edch = edch.transpose(1, 0, 2)

  xp = jnp.concatenate([x, jnp.ones((N, 1), jnp.float32)], axis=1)
  w1p = jnp.pad(W1, ((0, 0), (0, 1), (0, 0)))
  root1p = jnp.pad(root1, ((0, 1), (0, 0)))
  b1r = b1[None, :]
  b2r = b2[None, :]
  batch3d = batch.astype(jnp.int32).reshape(N // BN, 1, BN)

  a1p = _sc_agg1(xp, edch).reshape(NC, R, SEGS, 16)
  h, invr = _tc_layer1(a1p, xp, w1p, root1p, b1r)
  hflat = h.reshape(N * 16, 32)
  a2 = _sc_agg2(hflat, edch).reshape(R, SEGS, DH)
  return _tc_layer2(a2, invr, h, W2, root2, b2r, batch3d)
